# baseline (speedup 1.0000x reference)
import functools

import jax
import jax.numpy as jnp
from jax import lax
from jax.experimental import pallas as pl
from jax.experimental.pallas import tpu as pltpu
from jax.experimental.pallas import tpu_sc as plsc

NUM_CORES = 2
NUM_SUBCORES = 16
NUM_WORKERS = NUM_CORES * NUM_SUBCORES
LANES = 16

BATCH = 16384
RANK = 16
BPW = BATCH // NUM_WORKERS
CHUNK = 128
NCHUNK = BPW // CHUNK
NGROUP = BPW // LANES


def _bpr_body(uid_hbm, pid_hbm, nid_hbm, utab_hbm, itab_hbm,
              pos_hbm, neg_hbm,
              uid_v, pid_v, nid_v, urow_v, prow_v, nrow_v,
              pprod_v, nprod_v, pos_v, neg_v, sem):
    c = lax.axis_index("c")
    s = lax.axis_index("s")
    wid = s * NUM_CORES + c
    base = wid * BPW

    for j in range(NCHUNK):
        off = base + j * CHUNK
        pltpu.sync_copy(uid_hbm.at[pl.ds(off, CHUNK)], uid_v.at[j])
        pltpu.sync_copy(pid_hbm.at[pl.ds(off, CHUNK)], pid_v.at[j])
        pltpu.sync_copy(nid_hbm.at[pl.ds(off, CHUNK)], nid_v.at[j])

    copies = []
    for j in range(NCHUNK):
        dst = pl.ds(j * CHUNK, CHUNK)
        copies.append(pltpu.async_copy(utab_hbm.at[uid_v.at[j]], urow_v.at[dst], sem))
        copies.append(pltpu.async_copy(itab_hbm.at[pid_v.at[j]], prow_v.at[dst], sem))
        copies.append(pltpu.async_copy(itab_hbm.at[nid_v.at[j]], nrow_v.at[dst], sem))
    for cp in copies:
        cp.wait()

    iota = lax.iota(jnp.int32, LANES)

    def group(g, carry):
        row0 = g * LANES
        for i in range(LANES):
            u = urow_v[row0 + i, :]
            pprod_v[pl.ds(i * RANK, RANK)] = u * prow_v[row0 + i, :]
            nprod_v[pl.ds(i * RANK, RANK)] = u * nrow_v[row0 + i, :]
        accp = jnp.zeros((LANES,), jnp.float32)
        accn = jnp.zeros((LANES,), jnp.float32)
        for j in range(RANK):
            idx = iota * RANK + j
            accp = accp + plsc.load_gather(pprod_v, [idx])
            accn = accn + plsc.load_gather(nprod_v, [idx])
        pos_v[pl.ds(row0, LANES)] = accp
        neg_v[pl.ds(row0, LANES)] = accn
        return carry

    lax.fori_loop(0, NGROUP, group, 0)

    pltpu.sync_copy(pos_v, pos_hbm.at[pl.ds(base, BPW)])
    pltpu.sync_copy(neg_v, neg_hbm.at[pl.ds(base, BPW)])


@functools.partial(
    pl.kernel,
    out_type=(jax.ShapeDtypeStruct((BATCH,), jnp.float32),
              jax.ShapeDtypeStruct((BATCH,), jnp.float32)),
    mesh=plsc.VectorSubcoreMesh(core_axis_name="c", subcore_axis_name="s"),
    scratch_types=[
        pltpu.VMEM((NCHUNK, CHUNK), jnp.int32),
        pltpu.VMEM((NCHUNK, CHUNK), jnp.int32),
        pltpu.VMEM((NCHUNK, CHUNK), jnp.int32),
        pltpu.VMEM((BPW, RANK), jnp.float32),
        pltpu.VMEM((BPW, RANK), jnp.float32),
        pltpu.VMEM((BPW, RANK), jnp.float32),
        pltpu.VMEM((LANES * RANK,), jnp.float32),
        pltpu.VMEM((LANES * RANK,), jnp.float32),
        pltpu.VMEM((BPW,), jnp.float32),
        pltpu.VMEM((BPW,), jnp.float32),
        pltpu.SemaphoreType.DMA,
    ],
    compiler_params=pltpu.CompilerParams(needs_layout_passes=False,
                                         use_tc_tiling_on_sc=False),
)
def _bpr_sc(uid, pid, nid, utab, itab, pos_out, neg_out, *scratch):
    _bpr_body(uid, pid, nid, utab, itab, pos_out, neg_out, *scratch)


def kernel(user_ids, pos_items, neg_items, user_table, item_table):
    return _bpr_sc(user_ids.astype(jnp.int32),
                   pos_items.astype(jnp.int32),
                   neg_items.astype(jnp.int32),
                   user_table, item_table)
